# baseline (speedup 1.0000x reference)
import jax
import jax.numpy as jnp
from jax.experimental import pallas as pl
from jax.experimental.pallas import tpu as pltpu

NS = 100
NSP = 104
NP = 128
NB = 64
HID = 64
EH = 32
NL = 4
CPAD = NP - NS


def _ts(t):
    return t * (jnp.tanh(t) + 1.0)


G = 4


def _body(h_ref, eiw, eib, wsrc4, wdst, w2blk, w2, e1b4, e2b4, e2b,
          n1x, n1a, n1b, n2, n2b, eow, eob, out_ref):
    f32 = jnp.float32
    dot = lambda a, b: jnp.dot(a, b, preferred_element_type=f32)
    xs = [dot(h_ref[g], eiw[...]) + eib[...] for g in range(G)]
    for i in range(NL):
        a4s = [dot(x, wsrc4[i]) + e1b4[i] for x in xs]
        bs = [dot(x, wdst[i]) for x in xs]
        bps = []
        for b in bs:
            bz = jnp.pad(b, ((0, CPAD), (0, 0)))
            bps.append(jnp.concatenate(
                [bz[0:32], bz[32:64], bz[64:96], bz[96:128]], axis=1))
        a4ps = [jnp.pad(a4, ((0, NSP - NS), (0, 0))) for a4 in a4s]
        e1s = [_ts(bp[:, None, :] + a4p[None, :, :]).reshape(32 * NSP, 128)
               for bp, a4p in zip(bps, a4ps)]
        e2s = [_ts(dot(e1, w2blk[i]) + e2b4[i]) for e1 in e1s]
        aggs = []
        for e2 in e2s:
            s = e2.reshape(32, NSP, 128).sum(axis=0)[0:NS]
            aggs.append(s[:, 0:32] + s[:, 32:64] + s[:, 64:96] + s[:, 96:128])
        for g in range(G):
            ab = a4s[g][:, 0:32]
            t_pad = _ts(dot(_ts(ab), w2[i]) + e2b[i])
            t_diag = _ts(dot(_ts(ab + bs[g]), w2[i]) + e2b[i])
            aggs[g] = aggs[g] - float(CPAD) * t_pad - t_diag
        ms = [_ts(dot(x, n1x[i]) + dot(agg, n1a[i]) + n1b[i])
              for x, agg in zip(xs, aggs)]
        xs = [x + dot(m, n2[i]) + n2b[i] for x, m in zip(xs, ms)]
    for g in range(G):
        out_ref[g] = dot(xs[g], eow[...]) + eob[...]


@jax.jit
def kernel(h, emb_in_w, emb_in_b, ew1, eb1, ew2, eb2, nw1, nb1, nw2, nb2,
           emb_out_w, emb_out_b, rows, cols):
    f32 = jnp.float32
    wsrc = 0.5 * ew1[:, :HID, :]
    wdst = 0.5 * ew1[:, HID:, :]
    wsrc4 = jnp.concatenate([wsrc] * 4, axis=-1)
    e1b4 = 0.5 * jnp.tile(eb1, (1, 4))[:, None, :]
    e2b4 = 0.5 * jnp.tile(eb2, (1, 4))[:, None, :]
    w2blk = jax.vmap(lambda w: jnp.kron(jnp.eye(4, dtype=f32), 0.5 * w))(ew2)
    args = (h, emb_in_w, emb_in_b[None, :], wsrc4, wdst, w2blk, 0.5 * ew2,
            e1b4, e2b4, 0.5 * eb2[:, None, :], 0.5 * nw1[:, :HID, :],
            0.5 * nw1[:, HID:, :], 0.5 * nb1[:, None, :],
            nw2, nb2[:, None, :], emb_out_w, emb_out_b[None, :])

    def wspec(a):
        nd = a.ndim
        return pl.BlockSpec(a.shape, lambda i: (0,) * nd)

    in_specs = [pl.BlockSpec((G, NS, HID), lambda i: (i, 0, 0))]
    in_specs += [wspec(a) for a in args[1:]]
    out = pl.pallas_call(
        _body,
        grid=(NB // G,),
        in_specs=in_specs,
        out_specs=pl.BlockSpec((G, NS, HID), lambda i: (i, 0, 0)),
        out_shape=jax.ShapeDtypeStruct((NB, NS, HID), f32),
        compiler_params=pltpu.CompilerParams(
            dimension_semantics=("parallel",)),
    )(*args)
    return out.reshape(NB * NS, HID)


# speedup vs baseline: 1.1473x; 1.1473x over previous
import jax
import jax.numpy as jnp
from jax.experimental import pallas as pl
from jax.experimental.pallas import tpu as pltpu

NS = 100
NSP = 104
NP = 128
NB = 64
HID = 64
EH = 32
NL = 4
CPAD = NP - NS


def _ts(t):
    return t * jnp.tanh(t) + t


G = 4


def _body(h_ref, eiw, eib, wsrc4, wdst, w2blk, w2, e1b4, e2b4, e2b,
          n1x, n1a, n1b, n2, n2b, eow, eob, out_ref):
    f32 = jnp.float32
    dot = lambda a, b: jnp.dot(a, b, preferred_element_type=f32)
    xs = [dot(h_ref[g], eiw[...]) + eib[...] for g in range(G)]
    for i in range(NL):
        a4s = [dot(x, wsrc4[i]) + e1b4[i] for x in xs]
        bs = [dot(x, wdst[i]) for x in xs]
        bps = [jnp.concatenate(
            [b[0:25], b[25:50], b[50:75], b[75:100]], axis=1) for b in bs]
        a4ps = [jnp.pad(a4, ((0, NSP - NS), (0, 0))) for a4 in a4s]
        e1s = [_ts(bp[:, None, :] + a4p[None, :, :]).reshape(25 * NSP, 128)
               for bp, a4p in zip(bps, a4ps)]
        e2s = [_ts(dot(e1, w2blk[i]) + e2b4[i]) for e1 in e1s]
        aggs = []
        for e2 in e2s:
            s = e2.reshape(25, NSP, 128).sum(axis=0)[0:NS]
            aggs.append(s[:, 0:32] + s[:, 32:64] + s[:, 64:96] + s[:, 96:128])
        for g in range(G):
            ab = a4s[g][:, 0:32]
            t_diag = _ts(dot(_ts(ab + bs[g]), w2[i]) + e2b[i])
            aggs[g] = aggs[g] - t_diag
        ms = [_ts(dot(x, n1x[i]) + dot(agg, n1a[i]) + n1b[i])
              for x, agg in zip(xs, aggs)]
        xs = [x + dot(m, n2[i]) + n2b[i] for x, m in zip(xs, ms)]
    for g in range(G):
        out_ref[g] = dot(xs[g], eow[...]) + eob[...]


@jax.jit
def kernel(h, emb_in_w, emb_in_b, ew1, eb1, ew2, eb2, nw1, nb1, nw2, nb2,
           emb_out_w, emb_out_b, rows, cols):
    f32 = jnp.float32
    wsrc = 0.5 * ew1[:, :HID, :]
    wdst = 0.5 * ew1[:, HID:, :]
    wsrc4 = jnp.concatenate([wsrc] * 4, axis=-1)
    e1b4 = 0.5 * jnp.tile(eb1, (1, 4))[:, None, :]
    e2b4 = 0.5 * jnp.tile(eb2, (1, 4))[:, None, :]
    w2blk = jax.vmap(lambda w: jnp.kron(jnp.eye(4, dtype=f32), 0.5 * w))(ew2)
    args = (h, emb_in_w, emb_in_b[None, :], wsrc4, wdst, w2blk, 0.5 * ew2,
            e1b4, e2b4, 0.5 * eb2[:, None, :], 0.5 * nw1[:, :HID, :],
            0.5 * nw1[:, HID:, :], 0.5 * nb1[:, None, :],
            nw2, nb2[:, None, :], emb_out_w, emb_out_b[None, :])

    def wspec(a):
        nd = a.ndim
        return pl.BlockSpec(a.shape, lambda i: (0,) * nd)

    in_specs = [pl.BlockSpec((G, NS, HID), lambda i: (i, 0, 0))]
    in_specs += [wspec(a) for a in args[1:]]
    out = pl.pallas_call(
        _body,
        grid=(NB // G,),
        in_specs=in_specs,
        out_specs=pl.BlockSpec((G, NS, HID), lambda i: (i, 0, 0)),
        out_shape=jax.ShapeDtypeStruct((NB, NS, HID), f32),
        compiler_params=pltpu.CompilerParams(
            dimension_semantics=("parallel",)),
    )(*args)
    return out.reshape(NB * NS, HID)
